# baseline (speedup 1.0000x reference)
import functools

import jax
import jax.numpy as jnp
from jax import lax
from jax.experimental import pallas as pl
from jax.experimental.pallas import tpu as pltpu
from jax.experimental.pallas import tpu_sc as plsc

SEQ = 32768
VOCAB = 5
D = 384
LANES = 16

SEQ_SC = SEQ // 2
NC = 2
NS = 16
NW = NC * NS
BPW = SEQ_SC // NW
C = 128
NCHUNK = BPW // C
NBUF = 2

_mesh = plsc.VectorSubcoreMesh(core_axis_name="c", subcore_axis_name="s")


@functools.partial(
    pl.kernel,
    mesh=_mesh,
    compiler_params=pltpu.CompilerParams(needs_layout_passes=False),
    out_type=jax.ShapeDtypeStruct((SEQ_SC, D), jnp.float32),
    scratch_types=[
        pltpu.VMEM((BPW,), jnp.int32),
        pltpu.VMEM((NBUF, C, D), jnp.float32),
        pltpu.VMEM((VOCAB * D,), jnp.float32),
        pltpu.SemaphoreType.DMA,
    ],
)
def _sc_embed(ids_hbm, table_hbm, out_hbm, idx_v, rows_v, table_v, wsem):
    wid = lax.axis_index("s") * NC + lax.axis_index("c")
    base = wid * BPW

    pltpu.sync_copy(table_hbm, table_v)
    pltpu.sync_copy(ids_hbm.at[pl.ds(base, BPW)], idx_v)

    coliota = lax.iota(jnp.int32, LANES)

    def drain(b):
        pltpu.make_async_copy(
            rows_v.at[b], out_hbm.at[pl.ds(base, C)], wsem).wait()

    @pl.loop(0, NCHUNK, step=NBUF)
    def _round(k0):
        for b in range(NBUF):
            k = k0 + b

            @pl.when(k0 > 0)
            def _():
                drain(b)

            buf = rows_v.at[b]

            @plsc.parallel_loop(0, C, unroll=2)
            def _row(i, buf=buf, k=k):
                r = lax.rem(i, LANES)
                g16 = i - r
                ids16 = idx_v[pl.ds(k * C + g16, LANES)]
                rid16 = jnp.take(ids16, jnp.full((LANES,), 0, jnp.int32) + r)
                off = rid16 * D + coliota
                vals = [
                    plsc.load_gather(
                        table_v.at[pl.ds(j * LANES, VOCAB * D - j * LANES)],
                        [off])
                    for j in range(D // LANES)
                ]
                for j in range(D // LANES):
                    buf[i, pl.ds(j * LANES, LANES)] = vals[j]

            pltpu.async_copy(buf, out_hbm.at[pl.ds(base + k * C, C)], wsem)

    for b in range(NBUF):
        drain(b)


VPAD = 8
RB = 512
SEQ_TC = SEQ - SEQ_SC
NBLK = SEQ_TC // RB


def _tc_body(ids_ref, table_ref, out_ref):
    idb = ids_ref[0, 0, :]
    onehot = (idb[:, None] == lax.broadcasted_iota(jnp.int32, (RB, VPAD), 1))
    out_ref[...] = jnp.dot(onehot.astype(jnp.float32), table_ref[...],
                           preferred_element_type=jnp.float32)


_tc_expand = pl.pallas_call(
    _tc_body,
    grid=(NBLK,),
    in_specs=[
        pl.BlockSpec((1, 1, RB), lambda i: (i, 0, 0)),
        pl.BlockSpec((VPAD, D), lambda i: (0, 0)),
    ],
    out_specs=pl.BlockSpec((RB, D), lambda i: (i, 0)),
    out_shape=jax.ShapeDtypeStruct((SEQ_TC, D), jnp.float32),
    compiler_params=pltpu.CompilerParams(
        dimension_semantics=("parallel",)),
)


def kernel(ids, table):
    ids = ids.astype(jnp.int32)
    out_sc = _sc_embed(ids[:SEQ_SC], table.reshape(VOCAB * D))
    ids3 = ids[SEQ_SC:].reshape(NBLK, 1, RB)
    tpad = jnp.pad(table, ((0, VPAD - VOCAB), (0, 0)))
    out_tc = _tc_expand(ids3, tpad)
    return jnp.concatenate([out_sc, out_tc], axis=0)


# speedup vs baseline: 1.7985x; 1.7985x over previous
import functools

import jax
import jax.numpy as jnp
from jax import lax
from jax.experimental import pallas as pl
from jax.experimental.pallas import tpu as pltpu
from jax.experimental.pallas import tpu_sc as plsc

SEQ = 32768
VOCAB = 5
D = 384
LANES = 16
NC = 2
NS = 16
NW = NC * NS
BPW = SEQ // NW
C = 128
NCHUNK = BPW // C
NBUF = 2

_mesh = plsc.VectorSubcoreMesh(core_axis_name="c", subcore_axis_name="s")


@functools.partial(
    pl.kernel,
    mesh=_mesh,
    compiler_params=pltpu.CompilerParams(needs_layout_passes=False),
    out_type=jax.ShapeDtypeStruct((SEQ, D), jnp.float32),
    scratch_types=[
        pltpu.VMEM((BPW,), jnp.int32),
        pltpu.VMEM((NBUF, C, D), jnp.float32),
        pltpu.VMEM((VOCAB * D,), jnp.float32),
        pltpu.SemaphoreType.DMA,
    ],
)
def _embed(ids_hbm, table_hbm, out_hbm, idx_v, rows_v, table_v, wsem):
    wid = lax.axis_index("s") * NC + lax.axis_index("c")
    base = wid * BPW

    pltpu.sync_copy(table_hbm, table_v)
    pltpu.sync_copy(ids_hbm.at[pl.ds(base, BPW)], idx_v)

    coliota = lax.iota(jnp.int32, LANES)

    def drain(b):
        pltpu.make_async_copy(
            rows_v.at[b], out_hbm.at[pl.ds(base, C)], wsem).wait()

    @pl.loop(0, NCHUNK, step=NBUF)
    def _round(k0):
        for b in range(NBUF):
            k = k0 + b

            @pl.when(k0 > 0)
            def _():
                drain(b)

            buf = rows_v.at[b]

            @plsc.parallel_loop(0, C, unroll=4)
            def _row(i, buf=buf, k=k):
                r = lax.rem(i, LANES)
                g16 = i - r
                ids16 = idx_v[pl.ds(k * C + g16, LANES)]
                rid16 = jnp.take(ids16, jnp.full((LANES,), 0, jnp.int32) + r)
                off = rid16 * D + coliota
                vals = [
                    plsc.load_gather(
                        table_v.at[pl.ds(j * LANES, VOCAB * D - j * LANES)],
                        [off])
                    for j in range(D // LANES)
                ]
                for j in range(D // LANES):
                    buf[i, pl.ds(j * LANES, LANES)] = vals[j]

            pltpu.async_copy(buf, out_hbm.at[pl.ds(base + k * C, C)], wsem)

    for b in range(NBUF):
        drain(b)


def kernel(ids, table):
    return _embed(ids.astype(jnp.int32), table.reshape(VOCAB * D))


# speedup vs baseline: 2.0404x; 1.1345x over previous
import functools

import jax
import jax.numpy as jnp
from jax import lax
from jax.experimental import pallas as pl
from jax.experimental.pallas import tpu as pltpu
from jax.experimental.pallas import tpu_sc as plsc

SEQ = 32768
VOCAB = 5
D = 384
LANES = 16
NC = 2
NS = 16
NW = NC * NS
BPW = SEQ // NW
C = 64
NCHUNK = BPW // C
NBUF = 4

_mesh = plsc.VectorSubcoreMesh(core_axis_name="c", subcore_axis_name="s")


@functools.partial(
    pl.kernel,
    mesh=_mesh,
    compiler_params=pltpu.CompilerParams(needs_layout_passes=False),
    out_type=jax.ShapeDtypeStruct((SEQ, D), jnp.float32),
    scratch_types=[
        pltpu.VMEM((BPW,), jnp.int32),
        pltpu.VMEM((NBUF, C, D), jnp.float32),
        pltpu.VMEM((VOCAB * D,), jnp.float32),
        pltpu.SemaphoreType.DMA,
    ],
)
def _embed(ids_hbm, table_hbm, out_hbm, idx_v, rows_v, table_v, wsem):
    wid = lax.axis_index("s") * NC + lax.axis_index("c")
    base = wid * BPW

    pltpu.sync_copy(table_hbm, table_v)
    pltpu.sync_copy(ids_hbm.at[pl.ds(base, BPW)], idx_v)

    coliota = lax.iota(jnp.int32, LANES)

    def drain(b):
        pltpu.make_async_copy(
            rows_v.at[b], out_hbm.at[pl.ds(base, C)], wsem).wait()

    @pl.loop(0, NCHUNK, step=NBUF)
    def _round(k0):
        for b in range(NBUF):
            k = k0 + b

            @pl.when(k0 > 0)
            def _():
                drain(b)

            buf = rows_v.at[b]

            @plsc.parallel_loop(0, C, unroll=2)
            def _row(i, buf=buf, k=k):
                r = lax.rem(i, LANES)
                g16 = i - r
                ids16 = idx_v[pl.ds(k * C + g16, LANES)]
                rid16 = jnp.take(ids16, jnp.full((LANES,), 0, jnp.int32) + r)
                off = rid16 * D + coliota
                vals = [
                    plsc.load_gather(
                        table_v.at[pl.ds(j * LANES, VOCAB * D - j * LANES)],
                        [off])
                    for j in range(D // LANES)
                ]
                for j in range(D // LANES):
                    buf[i, pl.ds(j * LANES, LANES)] = vals[j]

            pltpu.async_copy(buf, out_hbm.at[pl.ds(base + k * C, C)], wsem)

    for b in range(NBUF):
        drain(b)


def kernel(ids, table):
    return _embed(ids.astype(jnp.int32), table.reshape(VOCAB * D))


# speedup vs baseline: 2.0676x; 1.0133x over previous
import functools

import jax
import jax.numpy as jnp
from jax import lax
from jax.experimental import pallas as pl
from jax.experimental.pallas import tpu as pltpu
from jax.experimental.pallas import tpu_sc as plsc

SEQ = 32768
VOCAB = 5
D = 384
LANES = 16
NC = 2
NS = 16
NW = NC * NS
BPW = SEQ // NW
C = 64
NCHUNK = BPW // C
NBUF = 4

_mesh = plsc.VectorSubcoreMesh(core_axis_name="c", subcore_axis_name="s")


@functools.partial(
    pl.kernel,
    mesh=_mesh,
    compiler_params=pltpu.CompilerParams(needs_layout_passes=False),
    out_type=jax.ShapeDtypeStruct((SEQ, D), jnp.float32),
    scratch_types=[
        pltpu.VMEM((BPW,), jnp.int32),
        pltpu.VMEM((NBUF, C, D), jnp.float32),
        pltpu.VMEM((VOCAB * D,), jnp.float32),
        pltpu.SemaphoreType.DMA,
    ],
)
def _embed(ids_hbm, table_hbm, out_hbm, idx_v, rows_v, table_v, wsem):
    wid = lax.axis_index("s") * NC + lax.axis_index("c")
    base = wid * BPW

    cp_tab = pltpu.async_copy(table_hbm, table_v, wsem)
    cp_ids = pltpu.async_copy(ids_hbm.at[pl.ds(base, BPW)], idx_v, wsem)
    cp_tab.wait()
    cp_ids.wait()

    coliota = lax.iota(jnp.int32, LANES)

    def drain(b):
        pltpu.make_async_copy(
            rows_v.at[b], out_hbm.at[pl.ds(base, C)], wsem).wait()

    @pl.loop(0, NCHUNK, step=NBUF)
    def _round(k0):
        for b in range(NBUF):
            k = k0 + b

            @pl.when(k0 > 0)
            def _():
                drain(b)

            buf = rows_v.at[b]

            @plsc.parallel_loop(0, C, unroll=2)
            def _row(i, buf=buf, k=k):
                r = lax.rem(i, LANES)
                g16 = i - r
                ids16 = idx_v[pl.ds(k * C + g16, LANES)]
                rid16 = jnp.take(ids16, jnp.full((LANES,), 0, jnp.int32) + r)
                off = rid16 * D + coliota
                vals = [
                    plsc.load_gather(
                        table_v.at[pl.ds(j * LANES, VOCAB * D - j * LANES)],
                        [off])
                    for j in range(D // LANES)
                ]
                for j in range(D // LANES):
                    buf[i, pl.ds(j * LANES, LANES)] = vals[j]

            pltpu.async_copy(buf, out_hbm.at[pl.ds(base + k * C, C)], wsem)

    for b in range(NBUF):
        drain(b)


def kernel(ids, table):
    return _embed(ids.astype(jnp.int32), table.reshape(VOCAB * D))
